# Initial kernel scaffold; baseline (speedup 1.0000x reference)
import jax
import jax.numpy as jnp
from jax.experimental import pallas as pl


def kernel(inputs, token_table, pos_table):
    raise NotImplementedError("write your pallas kernel here")


# speedup vs baseline: 1.4260x; 1.4260x over previous
import functools

import jax
import jax.numpy as jnp
from jax import lax
from jax.experimental import pallas as pl
from jax.experimental.pallas import tpu as pltpu
from jax.experimental.pallas import tpu_sc as plsc

VOCAB = 1000000
SEQ_LEN = 200
EMBED_DIM = 32
BATCH = 4096

NC = 2
NS = 16
NW = NC * NS

TOTAL = BATCH * SEQ_LEN
PER_W = TOTAL // NW
CHUNK = 1600
NCHUNKS = PER_W // CHUNK
REPS = CHUNK // SEQ_LEN

_UNITS = []
_o = 0
while _o < CHUNK:
    _u = min(128, CHUNK - _o)
    _UNITS.append((_o, _u))
    _o += _u


def _body(idx_hbm, tok_hbm, pos_hbm, out_hbm, idx_v, rows_v, pos_v, sem_g):
    wid = lax.axis_index("s") * NC + lax.axis_index("c")
    base = wid * PER_W
    pltpu.sync_copy(pos_hbm, pos_v)

    def chunk(g, carry):
        off = base + g * CHUNK
        pltpu.sync_copy(idx_hbm.at[pl.ds(off, CHUNK)], idx_v)
        cps = [
            pltpu.async_copy(
                tok_hbm.at[idx_v.at[pl.ds(u0, ul)]],
                rows_v.at[pl.ds(u0, ul)],
                sem_g,
            )
            for (u0, ul) in _UNITS
        ]
        for cp in cps:
            cp.wait()

        def posrow(p, c):
            pv0 = pos_v[p, pl.ds(0, 16)]
            pv1 = pos_v[p, pl.ds(16, 16)]
            for rep in range(REPS):
                r = rep * SEQ_LEN + p
                plsc.addupdate(rows_v.at[r, pl.ds(0, 16)], pv0)
                plsc.addupdate(rows_v.at[r, pl.ds(16, 16)], pv1)
            return c

        lax.fori_loop(0, SEQ_LEN, posrow, 0)
        pltpu.sync_copy(rows_v, out_hbm.at[pl.ds(off, CHUNK)])
        return carry

    lax.fori_loop(0, NCHUNKS, chunk, 0)


@functools.partial(jax.jit, static_argnames=())
def _run(idx_flat, token_table, pos_table):
    mesh = plsc.VectorSubcoreMesh(
        core_axis_name="c", subcore_axis_name="s", num_cores=NC, num_subcores=NS
    )
    return pl.kernel(
        _body,
        out_type=jax.ShapeDtypeStruct((TOTAL, EMBED_DIM), jnp.float32),
        mesh=mesh,
        scratch_types=[
            pltpu.VMEM((CHUNK,), jnp.int32),
            pltpu.VMEM((CHUNK, EMBED_DIM), jnp.float32),
            pltpu.VMEM((SEQ_LEN, EMBED_DIM), jnp.float32),
            pltpu.SemaphoreType.DMA,
        ],
        compiler_params=pltpu.CompilerParams(use_tc_tiling_on_sc=False),
    )(idx_flat, token_table, pos_table)


def kernel(inputs, token_table, pos_table):
    idx_flat = inputs.reshape(-1).astype(jnp.int32)
    out = _run(idx_flat, token_table, pos_table)
    return out.reshape(BATCH, SEQ_LEN, EMBED_DIM)


# speedup vs baseline: 1.4925x; 1.0466x over previous
import functools

import jax
import jax.numpy as jnp
from jax import lax
from jax.experimental import pallas as pl
from jax.experimental.pallas import tpu as pltpu
from jax.experimental.pallas import tpu_sc as plsc

VOCAB = 1000000
SEQ_LEN = 200
EMBED_DIM = 32
BATCH = 4096

NC = 2
NS = 16
NW = NC * NS

TOTAL = BATCH * SEQ_LEN
PER_W = TOTAL // NW
CHUNK = 1600
NCHUNKS = PER_W // CHUNK
REPS = CHUNK // SEQ_LEN

_UNITS = []
_o = 0
while _o < CHUNK:
    _u = min(128, CHUNK - _o)
    _UNITS.append((_o, _u))
    _o += _u


def _body(idx_hbm, tok_hbm, pos_hbm, out_hbm,
          idx0, idx1, rows0, rows1, pos_v,
          si0, si1, sg0, sg1, ss0, ss1):
    wid = lax.axis_index("s") * NC + lax.axis_index("c")
    base = wid * PER_W
    pltpu.sync_copy(pos_hbm, pos_v)

    idxs, rowss = (idx0, idx1), (rows0, rows1)
    sis, sgs, sss = (si0, si1), (sg0, sg1), (ss0, ss1)

    def start_idx(g, b):
        off = jnp.minimum(base + g * CHUNK, TOTAL - CHUNK)
        pltpu.async_copy(idx_hbm.at[pl.ds(off, CHUNK)], idxs[b], sis[b])

    def wait_idx(b):
        pltpu.make_async_copy(idx_hbm.at[pl.ds(0, CHUNK)], idxs[b], sis[b]).wait()

    def fire_gather(b):
        for u0, ul in _UNITS:
            pltpu.async_copy(
                tok_hbm.at[idxs[b].at[pl.ds(u0, ul)]],
                rowss[b].at[pl.ds(u0, ul)],
                sgs[b],
            )

    def wait_gather(b):
        pltpu.make_async_copy(tok_hbm.at[pl.ds(0, CHUNK)], rowss[b], sgs[b]).wait()

    def start_store(g, b):
        pltpu.async_copy(rowss[b], out_hbm.at[pl.ds(base + g * CHUNK, CHUNK)], sss[b])

    def wait_store(b):
        pltpu.make_async_copy(rowss[b], out_hbm.at[pl.ds(0, CHUNK)], sss[b]).wait()

    def add_pos(b):
        rv = rowss[b]

        def posrow(p, c):
            pv0 = pos_v[p, pl.ds(0, 16)]
            pv1 = pos_v[p, pl.ds(16, 16)]
            for rep in range(REPS):
                r = rep * SEQ_LEN + p
                plsc.addupdate(rv.at[r, pl.ds(0, 16)], pv0)
                plsc.addupdate(rv.at[r, pl.ds(16, 16)], pv1)
            return c

        lax.fori_loop(0, SEQ_LEN, posrow, 0)

    start_idx(0, 0)
    wait_idx(0)
    fire_gather(0)
    start_idx(1, 1)
    wait_idx(1)
    fire_gather(1)
    wait_gather(0)
    start_idx(2, 0)
    add_pos(0)
    start_store(0, 0)

    @pl.loop(1, NCHUNKS - 1, step=2)
    def _(g0):
        for b, g_off in ((1, 0), (0, 1)):
            g = g0 + g_off
            ob = 1 - b
            wait_idx(ob)
            wait_store(ob)
            fire_gather(ob)
            wait_gather(b)
            start_idx(g + 2, b)
            add_pos(b)
            start_store(g, b)

    wait_gather(1)
    add_pos(1)
    start_store(NCHUNKS - 1, 1)
    wait_idx(0)
    wait_store(0)
    wait_store(1)


@jax.jit
def _run(idx_flat, token_table, pos_table):
    mesh = plsc.VectorSubcoreMesh(
        core_axis_name="c", subcore_axis_name="s", num_cores=NC, num_subcores=NS
    )
    return pl.kernel(
        _body,
        out_type=jax.ShapeDtypeStruct((TOTAL, EMBED_DIM), jnp.float32),
        mesh=mesh,
        scratch_types=[
            pltpu.VMEM((CHUNK,), jnp.int32),
            pltpu.VMEM((CHUNK,), jnp.int32),
            pltpu.VMEM((CHUNK, EMBED_DIM), jnp.float32),
            pltpu.VMEM((CHUNK, EMBED_DIM), jnp.float32),
            pltpu.VMEM((SEQ_LEN, EMBED_DIM), jnp.float32),
            pltpu.SemaphoreType.DMA,
            pltpu.SemaphoreType.DMA,
            pltpu.SemaphoreType.DMA,
            pltpu.SemaphoreType.DMA,
            pltpu.SemaphoreType.DMA,
            pltpu.SemaphoreType.DMA,
        ],
        compiler_params=pltpu.CompilerParams(use_tc_tiling_on_sc=False),
    )(idx_flat, token_table, pos_table)


def kernel(inputs, token_table, pos_table):
    idx_flat = inputs.reshape(-1).astype(jnp.int32)
    out = _run(idx_flat, token_table, pos_table)
    return out.reshape(BATCH, SEQ_LEN, EMBED_DIM)
